# baseline (speedup 1.0000x reference)
import math

import jax
import jax.numpy as jnp
from jax.experimental import pallas as pl
from jax.experimental.pallas import tpu as pltpu

_TOPK = 13
_BETA = 6.0
_EPS = 1e-09
_DYN = 0.4


def _assigner_body(pd_ref, pb_ref, anc_ref, gtb_ref, gtl_ref, mgt_ref,
                   atp_ref, atg_ref, tl_ref, tb_ref, ts_ref, fg_ref,
                   tgi_ref):
    n = gtb_ref.shape[1]
    na, nc = pd_ref.shape[1], pd_ref.shape[2]
    f32 = jnp.float32
    hi = jax.lax.Precision.HIGHEST

    anc = anc_ref[...]
    ax, ay = anc[0:1, :], anc[1:2, :]
    gtb = gtb_ref[0]
    b1x1, b1y1, b1x2, b1y2 = gtb[:, 0:1], gtb[:, 1:2], gtb[:, 2:3], gtb[:, 3:4]
    mgt = mgt_ref[0]
    labi = gtl_ref[0]
    labc = labi.astype(f32)

    r_iota = jax.lax.broadcasted_iota(jnp.int32, (4, 1), 0)
    c4_iota = jax.lax.broadcasted_iota(jnp.int32, (1, 4), 1)
    eye4 = (r_iota == c4_iota).astype(f32)
    pb = jax.lax.dot_general(eye4, pb_ref[0], (((1,), (1,)), ((), ())),
                             precision=hi, preferred_element_type=f32)
    b2x1, b2y1, b2x2, b2y2 = pb[0:1, :], pb[1:2, :], pb[2:3, :], pb[3:4, :]

    c_iota = jax.lax.broadcasted_iota(jnp.int32, (1, nc), 1)
    oh_lab = (labi == c_iota).astype(f32)
    raw_scores = jax.lax.dot_general(oh_lab, pd_ref[0], (((1,), (1,)), ((), ())),
                                     precision=hi, preferred_element_type=f32)

    md = jnp.minimum(jnp.minimum(ax - b1x1, ay - b1y1),
                     jnp.minimum(b1x2 - ax, b1y2 - ay))
    mask_in = (md > 1e-09).astype(f32)
    valid = (mask_in * mgt) > 0

    bbox_scores = jnp.where(valid, raw_scores, 0.0)

    eps = 1e-7
    w1 = b1x2 - b1x1
    h1 = b1y2 - b1y1 + eps
    w2 = b2x2 - b2x1
    h2 = b2y2 - b2y1 + eps
    inter = (jnp.maximum(jnp.minimum(b1x2, b2x2) - jnp.maximum(b1x1, b2x1), 0.0) *
             jnp.maximum(jnp.minimum(b1y2, b2y2) - jnp.maximum(b1y1, b2y1), 0.0))
    union = w1 * h1 + w2 * h2 - inter + eps
    iou = inter / union
    cw = jnp.maximum(b1x2, b2x2) - jnp.minimum(b1x1, b2x1)
    ch = jnp.maximum(b1y2, b2y2) - jnp.minimum(b1y1, b2y1)
    c2 = cw ** 2 + ch ** 2 + eps
    rho2 = ((b2x1 + b2x2 - b1x1 - b1x2) ** 2 +
            (b2y1 + b2y2 - b1y1 - b1y2) ** 2) / 4.0
    v = (4.0 / math.pi ** 2) * (atp_ref[0] - atg_ref[0]) ** 2
    alpha = v / (v - iou + (1.0 + eps))
    ciou = iou - (rho2 / c2 + v * alpha)
    overlaps = jnp.where(valid, jnp.maximum(ciou, 0.0), 0.0)

    align = bbox_scores * overlaps ** _BETA

    a_iota = jax.lax.broadcasted_iota(jnp.int32, (1, na), 1)
    g_iota = jax.lax.broadcasted_iota(jnp.int32, (n, 1), 0)
    big = jnp.int32(1 << 30)
    work = align
    selm = jnp.zeros_like(align)
    for _ in range(_TOPK):
        m = jnp.max(work, axis=1, keepdims=True)
        idx = jnp.min(jnp.where(work == m, a_iota, big), axis=1, keepdims=True)
        pick = a_iota == idx
        selm = jnp.where(pick, 1.0, selm)
        work = jnp.where(pick, -1.0, work)

    mask_pos = selm * mask_in * mgt

    fg = jnp.sum(mask_pos, axis=0, keepdims=True)
    multi = fg > 1.0
    maxo = jnp.max(overlaps, axis=0, keepdims=True)
    amax_idx = jnp.min(jnp.where(overlaps == maxo, g_iota, big),
                       axis=0, keepdims=True)
    is_max = (g_iota == amax_idx).astype(f32)
    mask_pos = jnp.where(multi, is_max, mask_pos)
    fg = jnp.sum(mask_pos, axis=0, keepdims=True)
    tg = jnp.min(jnp.where(mask_pos > 0, g_iota, big), axis=0, keepdims=True)
    tg = jnp.where(tg == big, 0, tg)

    onehot_n = g_iota == tg
    tl = jnp.sum(jnp.where(onehot_n, labc, 0.0), axis=0, keepdims=True)
    tl_i = tl.astype(jnp.int32)
    tl_ref[0] = tl_i
    tgi_ref[0] = tg
    fg_ref[0] = fg
    tb_ref[0] = jnp.concatenate(
        [jnp.sum(jnp.where(onehot_n, gtb[:, j:j + 1], 0.0),
                 axis=0, keepdims=True) for j in range(4)], axis=0)

    am = align * mask_pos
    dyn = _DYN * jnp.max(am, axis=1, keepdims=True)
    norm = jnp.max(am / (dyn + _EPS), axis=0, keepdims=True)
    scale = jnp.where(fg > 0, norm, 0.0)
    tl_col = jnp.transpose(tl_i, (1, 0))
    scale_col = jnp.transpose(scale, (1, 0))
    ts_ref[0] = jnp.where(c_iota == tl_col, scale_col, 0.0)


def kernel(pd_scores, pd_bboxes, anc_points, gt_labels, gt_bboxes, mask_gt):
    bs, na, nc = pd_scores.shape
    n = gt_bboxes.shape[1]

    ancT = jnp.transpose(anc_points, (1, 0))

    eps = 1e-7
    atan_pd = jnp.arctan((pd_bboxes[..., 2] - pd_bboxes[..., 0]) /
                         (pd_bboxes[..., 3] - pd_bboxes[..., 1] + eps))
    atan_pd = atan_pd.reshape(bs, 1, na)
    atan_gt = jnp.arctan((gt_bboxes[..., 2] - gt_bboxes[..., 0]) /
                         (gt_bboxes[..., 3] - gt_bboxes[..., 1] + eps))
    atan_gt = atan_gt.reshape(bs, n, 1)

    out_shape = [
        jax.ShapeDtypeStruct((bs, 1, na), jnp.int32),
        jax.ShapeDtypeStruct((bs, 4, na), jnp.float32),
        jax.ShapeDtypeStruct((bs, na, nc), jnp.float32),
        jax.ShapeDtypeStruct((bs, 1, na), jnp.float32),
        jax.ShapeDtypeStruct((bs, 1, na), jnp.int32),
    ]
    tl3, tb3, ts3, fg3, tgi3 = pl.pallas_call(
        _assigner_body,
        grid=(bs,),
        in_specs=[
            pl.BlockSpec((1, na, nc), lambda b: (b, 0, 0)),
            pl.BlockSpec((1, na, 4), lambda b: (b, 0, 0)),
            pl.BlockSpec((2, na), lambda b: (0, 0)),
            pl.BlockSpec((1, n, 4), lambda b: (b, 0, 0)),
            pl.BlockSpec((1, n, 1), lambda b: (b, 0, 0)),
            pl.BlockSpec((1, n, 1), lambda b: (b, 0, 0)),
            pl.BlockSpec((1, 1, na), lambda b: (b, 0, 0)),
            pl.BlockSpec((1, n, 1), lambda b: (b, 0, 0)),
        ],
        out_specs=[
            pl.BlockSpec((1, 1, na), lambda b: (b, 0, 0)),
            pl.BlockSpec((1, 4, na), lambda b: (b, 0, 0)),
            pl.BlockSpec((1, na, nc), lambda b: (b, 0, 0)),
            pl.BlockSpec((1, 1, na), lambda b: (b, 0, 0)),
            pl.BlockSpec((1, 1, na), lambda b: (b, 0, 0)),
        ],
        out_shape=out_shape,
        compiler_params=pltpu.CompilerParams(
            dimension_semantics=("arbitrary",)),
    )(pd_scores, pd_bboxes, ancT, gt_bboxes, gt_labels, mask_gt,
      atan_pd, atan_gt)

    target_labels = tl3.reshape(bs, na)
    target_bboxes = jnp.transpose(tb3, (0, 2, 1))
    target_scores = ts3
    fg_mask = fg3.reshape(bs, na).astype(bool)
    target_gt_idx = tgi3.reshape(bs, na)
    return target_labels, target_bboxes, target_scores, fg_mask, target_gt_idx


# speedup vs baseline: 2.1107x; 2.1107x over previous
import math

import jax
import jax.numpy as jnp
from jax.experimental import pallas as pl
from jax.experimental.pallas import tpu as pltpu

_TOPK = 13
_BETA = 6.0
_EPS = 1e-09
_DYN = 0.4


def _assigner_body(lab_sm, pdT_ref, pbT_ref, anc_ref, gtb_ref, gtl_ref,
                   mgt_ref, atp_ref, atg_ref, tl_ref, tb_ref, ts_ref, fg_ref,
                   tgi_ref, sc_ref):
    b = pl.program_id(0)
    n, na = sc_ref.shape
    nc = ts_ref.shape[1]
    f32 = jnp.float32

    anc = anc_ref[...]
    ax, ay = anc[0:1, :], anc[1:2, :]
    pb = pbT_ref[0]
    b2x1, b2y1, b2x2, b2y2 = pb[0:1, :], pb[1:2, :], pb[2:3, :], pb[3:4, :]
    gtb = gtb_ref[0]
    b1x1, b1y1, b1x2, b1y2 = gtb[:, 0:1], gtb[:, 1:2], gtb[:, 2:3], gtb[:, 3:4]
    mgt = mgt_ref[0]
    labc = gtl_ref[0].astype(f32)

    def _gather(g, c):
        lab = lab_sm[b, g]
        sc_ref[pl.ds(g, 1), :] = pdT_ref[0, pl.ds(lab, 1), :]
        return c
    jax.lax.fori_loop(0, n, _gather, 0)

    md = jnp.minimum(jnp.minimum(ax - b1x1, ay - b1y1),
                     jnp.minimum(b1x2 - ax, b1y2 - ay))
    mask_in = (md > 1e-09).astype(f32)
    valid = (mask_in * mgt) > 0

    bbox_scores = jnp.where(valid, sc_ref[...], 0.0)

    eps = 1e-7
    w1 = b1x2 - b1x1
    h1 = b1y2 - b1y1 + eps
    w2 = b2x2 - b2x1
    h2 = b2y2 - b2y1 + eps
    inter = (jnp.maximum(jnp.minimum(b1x2, b2x2) - jnp.maximum(b1x1, b2x1), 0.0) *
             jnp.maximum(jnp.minimum(b1y2, b2y2) - jnp.maximum(b1y1, b2y1), 0.0))
    union = w1 * h1 + w2 * h2 - inter + eps
    iou = inter / union
    cw = jnp.maximum(b1x2, b2x2) - jnp.minimum(b1x1, b2x1)
    ch = jnp.maximum(b1y2, b2y2) - jnp.minimum(b1y1, b2y1)
    c2 = cw ** 2 + ch ** 2 + eps
    rho2 = ((b2x1 + b2x2 - b1x1 - b1x2) ** 2 +
            (b2y1 + b2y2 - b1y1 - b1y2) ** 2) / 4.0
    v = (4.0 / math.pi ** 2) * (atp_ref[0] - atg_ref[0]) ** 2
    alpha = v / (v - iou + (1.0 + eps))
    ciou = iou - (rho2 / c2 + v * alpha)
    overlaps = jnp.where(valid, jnp.maximum(ciou, 0.0), 0.0)

    align = bbox_scores * overlaps ** _BETA

    a_iota = jax.lax.broadcasted_iota(jnp.int32, (1, na), 1)
    g_iota = jax.lax.broadcasted_iota(jnp.int32, (n, 1), 0)
    big = jnp.int32(1 << 30)
    work = align
    selm = jnp.zeros_like(align)
    for _ in range(_TOPK):
        m = jnp.max(work, axis=1, keepdims=True)
        idx = jnp.min(jnp.where(work == m, a_iota, big), axis=1, keepdims=True)
        pick = a_iota == idx
        selm = jnp.where(pick, 1.0, selm)
        work = jnp.where(pick, -1.0, work)

    mask_pos = selm * mask_in * mgt

    fg = jnp.sum(mask_pos, axis=0, keepdims=True)
    multi = fg > 1.0
    amax_idx = jnp.argmax(overlaps, axis=0, keepdims=True).astype(jnp.int32)
    is_max = (g_iota == amax_idx).astype(f32)
    mask_pos = jnp.where(multi, is_max, mask_pos)
    fg = jnp.sum(mask_pos, axis=0, keepdims=True)
    tg = jnp.argmax(mask_pos, axis=0, keepdims=True).astype(jnp.int32)

    onehot_n = g_iota == tg
    tl = jnp.sum(jnp.where(onehot_n, labc, 0.0), axis=0, keepdims=True)
    tl_i = tl.astype(jnp.int32)
    tl_ref[0] = tl_i
    tgi_ref[0] = tg
    fg_ref[0] = fg
    tb_ref[0] = jnp.concatenate(
        [jnp.sum(jnp.where(onehot_n, gtb[:, j:j + 1], 0.0),
                 axis=0, keepdims=True) for j in range(4)], axis=0)

    am = align * mask_pos
    dyn = _DYN * jnp.max(am, axis=1, keepdims=True)
    norm = jnp.max(am / (dyn + _EPS), axis=0, keepdims=True)
    scale = jnp.where(fg > 0, norm, 0.0)
    c_iota = jax.lax.broadcasted_iota(jnp.int32, (nc, 1), 0)
    ts_ref[0] = jnp.where(c_iota == tl_i, scale, 0.0)


def kernel(pd_scores, pd_bboxes, anc_points, gt_labels, gt_bboxes, mask_gt):
    bs, na, nc = pd_scores.shape
    n = gt_bboxes.shape[1]

    pdT = jnp.transpose(pd_scores, (0, 2, 1))
    pbT = jnp.transpose(pd_bboxes, (0, 2, 1))
    ancT = jnp.transpose(anc_points, (1, 0))
    lab = gt_labels.reshape(bs, n)

    eps = 1e-7
    atan_pd = jnp.arctan((pd_bboxes[..., 2] - pd_bboxes[..., 0]) /
                         (pd_bboxes[..., 3] - pd_bboxes[..., 1] + eps))
    atan_pd = atan_pd.reshape(bs, 1, na)
    atan_gt = jnp.arctan((gt_bboxes[..., 2] - gt_bboxes[..., 0]) /
                         (gt_bboxes[..., 3] - gt_bboxes[..., 1] + eps))
    atan_gt = atan_gt.reshape(bs, n, 1)

    out_shape = [
        jax.ShapeDtypeStruct((bs, 1, na), jnp.int32),
        jax.ShapeDtypeStruct((bs, 4, na), jnp.float32),
        jax.ShapeDtypeStruct((bs, nc, na), jnp.float32),
        jax.ShapeDtypeStruct((bs, 1, na), jnp.float32),
        jax.ShapeDtypeStruct((bs, 1, na), jnp.int32),
    ]
    grid_spec = pltpu.PrefetchScalarGridSpec(
        num_scalar_prefetch=1,
        grid=(bs,),
        in_specs=[
            pl.BlockSpec((1, nc, na), lambda b, L: (b, 0, 0)),
            pl.BlockSpec((1, 4, na), lambda b, L: (b, 0, 0)),
            pl.BlockSpec((2, na), lambda b, L: (0, 0)),
            pl.BlockSpec((1, n, 4), lambda b, L: (b, 0, 0)),
            pl.BlockSpec((1, n, 1), lambda b, L: (b, 0, 0)),
            pl.BlockSpec((1, n, 1), lambda b, L: (b, 0, 0)),
            pl.BlockSpec((1, 1, na), lambda b, L: (b, 0, 0)),
            pl.BlockSpec((1, n, 1), lambda b, L: (b, 0, 0)),
        ],
        out_specs=[
            pl.BlockSpec((1, 1, na), lambda b, L: (b, 0, 0)),
            pl.BlockSpec((1, 4, na), lambda b, L: (b, 0, 0)),
            pl.BlockSpec((1, nc, na), lambda b, L: (b, 0, 0)),
            pl.BlockSpec((1, 1, na), lambda b, L: (b, 0, 0)),
            pl.BlockSpec((1, 1, na), lambda b, L: (b, 0, 0)),
        ],
        scratch_shapes=[pltpu.VMEM((n, na), jnp.float32)],
    )
    tl3, tb3, ts3, fg3, tgi3 = pl.pallas_call(
        _assigner_body,
        grid_spec=grid_spec,
        out_shape=out_shape,
        compiler_params=pltpu.CompilerParams(
            dimension_semantics=("arbitrary",)),
    )(lab, pdT, pbT, ancT, gt_bboxes, gt_labels, mask_gt, atan_pd, atan_gt)

    target_labels = tl3.reshape(bs, na)
    target_bboxes = jnp.transpose(tb3, (0, 2, 1))
    target_scores = jnp.transpose(ts3, (0, 2, 1))
    fg_mask = fg3.reshape(bs, na).astype(bool)
    target_gt_idx = tgi3.reshape(bs, na)
    return target_labels, target_bboxes, target_scores, fg_mask, target_gt_idx


# speedup vs baseline: 2.1572x; 1.0220x over previous
import math

import jax
import jax.numpy as jnp
from jax.experimental import pallas as pl
from jax.experimental.pallas import tpu as pltpu

_TOPK = 13
_BETA = 6.0
_EPS = 1e-09
_DYN = 0.4


def _assigner_body(lab_sm, pdT_ref, pbT_ref, anc_ref, gtb_ref, gtl_ref,
                   mgt_ref, atp_ref, atg_ref, tl_ref, tb_ref, ts_ref, fg_ref,
                   tgi_ref, sc_ref):
    b = pl.program_id(0)
    n, na = sc_ref.shape
    nc = ts_ref.shape[1]
    f32 = jnp.float32

    anc = anc_ref[...]
    ax, ay = anc[0:1, :], anc[1:2, :]
    pb = pbT_ref[0]
    b2x1, b2y1, b2x2, b2y2 = pb[0:1, :], pb[1:2, :], pb[2:3, :], pb[3:4, :]
    gtb = gtb_ref[0]
    b1x1, b1y1, b1x2, b1y2 = gtb[:, 0:1], gtb[:, 1:2], gtb[:, 2:3], gtb[:, 3:4]
    mgt = mgt_ref[0]
    labc = gtl_ref[0].astype(f32)

    for g in range(n):
        lab = lab_sm[b, g]
        sc_ref[pl.ds(g, 1), :] = pdT_ref[0, pl.ds(lab, 1), :]

    md = jnp.minimum(jnp.minimum(ax - b1x1, ay - b1y1),
                     jnp.minimum(b1x2 - ax, b1y2 - ay))
    mask_in = (md > 1e-09).astype(f32)
    valid = (mask_in * mgt) > 0

    bbox_scores = jnp.where(valid, sc_ref[...], 0.0)

    eps = 1e-7
    w1 = b1x2 - b1x1
    h1 = b1y2 - b1y1 + eps
    w2 = b2x2 - b2x1
    h2 = b2y2 - b2y1 + eps
    inter = (jnp.maximum(jnp.minimum(b1x2, b2x2) - jnp.maximum(b1x1, b2x1), 0.0) *
             jnp.maximum(jnp.minimum(b1y2, b2y2) - jnp.maximum(b1y1, b2y1), 0.0))
    union = w1 * h1 + w2 * h2 - inter + eps
    iou = inter / union
    cw = jnp.maximum(b1x2, b2x2) - jnp.minimum(b1x1, b2x1)
    ch = jnp.maximum(b1y2, b2y2) - jnp.minimum(b1y1, b2y1)
    c2 = cw ** 2 + ch ** 2 + eps
    rho2 = ((b2x1 + b2x2 - b1x1 - b1x2) ** 2 +
            (b2y1 + b2y2 - b1y1 - b1y2) ** 2) / 4.0
    v = (4.0 / math.pi ** 2) * (atp_ref[0] - atg_ref[0]) ** 2
    alpha = v / (v - iou + (1.0 + eps))
    ciou = iou - (rho2 / c2 + v * alpha)
    overlaps = jnp.where(valid, jnp.maximum(ciou, 0.0), 0.0)

    align = bbox_scores * overlaps ** _BETA

    a_iota = jax.lax.broadcasted_iota(jnp.int32, (1, na), 1)
    g_iota = jax.lax.broadcasted_iota(jnp.int32, (n, 1), 0)
    big = jnp.int32(1 << 30)
    work = align
    selm = jnp.zeros_like(align)
    for _ in range(_TOPK):
        m = jnp.max(work, axis=1, keepdims=True)
        idx = jnp.min(jnp.where(work == m, a_iota, big), axis=1, keepdims=True)
        pick = a_iota == idx
        selm = jnp.where(pick, 1.0, selm)
        work = jnp.where(pick, -1.0, work)

    mask_pos = selm * mask_in * mgt

    fg = jnp.sum(mask_pos, axis=0, keepdims=True)
    multi = fg > 1.0
    amax_idx = jnp.argmax(overlaps, axis=0, keepdims=True).astype(jnp.int32)
    is_max = (g_iota == amax_idx).astype(f32)
    mask_pos = jnp.where(multi, is_max, mask_pos)
    fg = jnp.sum(mask_pos, axis=0, keepdims=True)
    tg = jnp.argmax(mask_pos, axis=0, keepdims=True).astype(jnp.int32)

    onehot_n = g_iota == tg
    tl = jnp.sum(jnp.where(onehot_n, labc, 0.0), axis=0, keepdims=True)
    tl_i = tl.astype(jnp.int32)
    tl_ref[0] = tl_i
    tgi_ref[0] = tg
    fg_ref[0] = fg
    tb_ref[0] = jnp.concatenate(
        [jnp.sum(jnp.where(onehot_n, gtb[:, j:j + 1], 0.0),
                 axis=0, keepdims=True) for j in range(4)], axis=0)

    am = align * mask_pos
    dyn = _DYN * jnp.max(am, axis=1, keepdims=True)
    norm = jnp.max(am / (dyn + _EPS), axis=0, keepdims=True)
    scale = jnp.where(fg > 0, norm, 0.0)
    c_iota = jax.lax.broadcasted_iota(jnp.int32, (nc, 1), 0)
    ts_ref[0] = jnp.where(c_iota == tl_i, scale, 0.0)


def kernel(pd_scores, pd_bboxes, anc_points, gt_labels, gt_bboxes, mask_gt):
    bs, na, nc = pd_scores.shape
    n = gt_bboxes.shape[1]

    pdT = jnp.transpose(pd_scores, (0, 2, 1))
    pbT = jnp.transpose(pd_bboxes, (0, 2, 1))
    ancT = jnp.transpose(anc_points, (1, 0))
    lab = gt_labels.reshape(bs, n)

    eps = 1e-7
    atan_pd = jnp.arctan((pd_bboxes[..., 2] - pd_bboxes[..., 0]) /
                         (pd_bboxes[..., 3] - pd_bboxes[..., 1] + eps))
    atan_pd = atan_pd.reshape(bs, 1, na)
    atan_gt = jnp.arctan((gt_bboxes[..., 2] - gt_bboxes[..., 0]) /
                         (gt_bboxes[..., 3] - gt_bboxes[..., 1] + eps))
    atan_gt = atan_gt.reshape(bs, n, 1)

    out_shape = [
        jax.ShapeDtypeStruct((bs, 1, na), jnp.int32),
        jax.ShapeDtypeStruct((bs, 4, na), jnp.float32),
        jax.ShapeDtypeStruct((bs, nc, na), jnp.float32),
        jax.ShapeDtypeStruct((bs, 1, na), jnp.float32),
        jax.ShapeDtypeStruct((bs, 1, na), jnp.int32),
    ]
    grid_spec = pltpu.PrefetchScalarGridSpec(
        num_scalar_prefetch=1,
        grid=(bs,),
        in_specs=[
            pl.BlockSpec((1, nc, na), lambda b, L: (b, 0, 0)),
            pl.BlockSpec((1, 4, na), lambda b, L: (b, 0, 0)),
            pl.BlockSpec((2, na), lambda b, L: (0, 0)),
            pl.BlockSpec((1, n, 4), lambda b, L: (b, 0, 0)),
            pl.BlockSpec((1, n, 1), lambda b, L: (b, 0, 0)),
            pl.BlockSpec((1, n, 1), lambda b, L: (b, 0, 0)),
            pl.BlockSpec((1, 1, na), lambda b, L: (b, 0, 0)),
            pl.BlockSpec((1, n, 1), lambda b, L: (b, 0, 0)),
        ],
        out_specs=[
            pl.BlockSpec((1, 1, na), lambda b, L: (b, 0, 0)),
            pl.BlockSpec((1, 4, na), lambda b, L: (b, 0, 0)),
            pl.BlockSpec((1, nc, na), lambda b, L: (b, 0, 0)),
            pl.BlockSpec((1, 1, na), lambda b, L: (b, 0, 0)),
            pl.BlockSpec((1, 1, na), lambda b, L: (b, 0, 0)),
        ],
        scratch_shapes=[pltpu.VMEM((n, na), jnp.float32)],
    )
    tl3, tb3, ts3, fg3, tgi3 = pl.pallas_call(
        _assigner_body,
        grid_spec=grid_spec,
        out_shape=out_shape,
        compiler_params=pltpu.CompilerParams(
            dimension_semantics=("arbitrary",)),
    )(lab, pdT, pbT, ancT, gt_bboxes, gt_labels, mask_gt, atan_pd, atan_gt)

    target_labels = tl3.reshape(bs, na)
    target_bboxes = jnp.transpose(tb3, (0, 2, 1))
    target_scores = jnp.transpose(ts3, (0, 2, 1))
    fg_mask = fg3.reshape(bs, na).astype(bool)
    target_gt_idx = tgi3.reshape(bs, na)
    return target_labels, target_bboxes, target_scores, fg_mask, target_gt_idx


# speedup vs baseline: 2.1717x; 1.0067x over previous
import math

import jax
import jax.numpy as jnp
from jax.experimental import pallas as pl
from jax.experimental.pallas import tpu as pltpu

_TOPK = 13
_BETA = 6.0
_EPS = 1e-09
_DYN = 0.4


def _assigner_body(lab_sm, pdT_ref, pbT_ref, anc_ref, gtb_ref, gtl_ref,
                   mgt_ref, atp_ref, atg_ref, tl_ref, tb_ref, ts_ref, fg_ref,
                   tgi_ref, sc_ref):
    b = pl.program_id(0)
    n, na = sc_ref.shape
    nc = ts_ref.shape[1]
    f32 = jnp.float32

    anc = anc_ref[...]
    ax, ay = anc[0:1, :], anc[1:2, :]
    pb = pbT_ref[0]
    b2x1, b2y1, b2x2, b2y2 = pb[0:1, :], pb[1:2, :], pb[2:3, :], pb[3:4, :]
    gtb = gtb_ref[0]
    b1x1, b1y1, b1x2, b1y2 = gtb[:, 0:1], gtb[:, 1:2], gtb[:, 2:3], gtb[:, 3:4]
    mgt = mgt_ref[0]
    labc = gtl_ref[0].astype(f32)

    for g in range(n):
        lab = lab_sm[b, g]
        sc_ref[pl.ds(g, 1), :] = pdT_ref[0, pl.ds(lab, 1), :]

    md = jnp.minimum(jnp.minimum(ax - b1x1, ay - b1y1),
                     jnp.minimum(b1x2 - ax, b1y2 - ay))
    mask_in = (md > 1e-09).astype(f32)
    valid = (mask_in * mgt) > 0

    bbox_scores = jnp.where(valid, sc_ref[...], 0.0)

    eps = 1e-7
    w1 = b1x2 - b1x1
    h1 = b1y2 - b1y1 + eps
    w2 = b2x2 - b2x1
    h2 = b2y2 - b2y1 + eps
    inter = (jnp.maximum(jnp.minimum(b1x2, b2x2) - jnp.maximum(b1x1, b2x1), 0.0) *
             jnp.maximum(jnp.minimum(b1y2, b2y2) - jnp.maximum(b1y1, b2y1), 0.0))
    union = w1 * h1 + w2 * h2 - inter + eps
    iou = inter / union
    cw = jnp.maximum(b1x2, b2x2) - jnp.minimum(b1x1, b2x1)
    ch = jnp.maximum(b1y2, b2y2) - jnp.minimum(b1y1, b2y1)
    c2 = cw ** 2 + ch ** 2 + eps
    rho2 = ((b2x1 + b2x2 - b1x1 - b1x2) ** 2 +
            (b2y1 + b2y2 - b1y1 - b1y2) ** 2) / 4.0
    v = (4.0 / math.pi ** 2) * (atp_ref[0] - atg_ref[0]) ** 2
    alpha = v / (v - iou + (1.0 + eps))
    ciou = iou - (rho2 / c2 + v * alpha)
    overlaps = jnp.where(valid, jnp.maximum(ciou, 0.0), 0.0)

    align = bbox_scores * overlaps ** _BETA

    g_iota = jax.lax.broadcasted_iota(jnp.int32, (n, 1), 0)
    big = jnp.int32(1 << 30)
    pad = (-(-na // 128)) * 128 - na
    work = jnp.concatenate(
        [align, jnp.full((n, pad), -2.0, f32)], axis=1)
    nap = na + pad
    a_iota = jax.lax.broadcasted_iota(jnp.int32, (1, nap), 1)
    l_iota = jax.lax.broadcasted_iota(jnp.int32, (n, 128), 1)
    selm = jnp.zeros((n, nap), f32)
    for _ in range(_TOPK):
        mv = work[:, 0:128]
        mi = l_iota
        for t in range(1, nap // 128):
            wt = work[:, t * 128:(t + 1) * 128]
            better = wt > mv
            mv = jnp.where(better, wt, mv)
            mi = jnp.where(better, l_iota + t * 128, mi)
        m = jnp.max(mv, axis=1, keepdims=True)
        idx = jnp.min(jnp.where(mv == m, mi, big), axis=1, keepdims=True)
        pick = a_iota == idx
        selm = jnp.where(pick, 1.0, selm)
        work = jnp.where(pick, -1.0, work)
    selm = selm[:, :na]

    mask_pos = selm * mask_in * mgt

    fg = jnp.sum(mask_pos, axis=0, keepdims=True)
    multi = fg > 1.0
    amax_idx = jnp.argmax(overlaps, axis=0, keepdims=True).astype(jnp.int32)
    is_max = (g_iota == amax_idx).astype(f32)
    mask_pos = jnp.where(multi, is_max, mask_pos)
    fg = jnp.sum(mask_pos, axis=0, keepdims=True)
    tg = jnp.argmax(mask_pos, axis=0, keepdims=True).astype(jnp.int32)

    onehot_n = g_iota == tg
    tl = jnp.sum(jnp.where(onehot_n, labc, 0.0), axis=0, keepdims=True)
    tl_i = tl.astype(jnp.int32)
    tl_ref[0] = tl_i
    tgi_ref[0] = tg
    fg_ref[0] = fg
    tb_ref[0] = jnp.concatenate(
        [jnp.sum(jnp.where(onehot_n, gtb[:, j:j + 1], 0.0),
                 axis=0, keepdims=True) for j in range(4)], axis=0)

    am = align * mask_pos
    dyn = _DYN * jnp.max(am, axis=1, keepdims=True)
    norm = jnp.max(am / (dyn + _EPS), axis=0, keepdims=True)
    scale = jnp.where(fg > 0, norm, 0.0)
    c_iota = jax.lax.broadcasted_iota(jnp.int32, (nc, 1), 0)
    ts_ref[0] = jnp.where(c_iota == tl_i, scale, 0.0)


def kernel(pd_scores, pd_bboxes, anc_points, gt_labels, gt_bboxes, mask_gt):
    bs, na, nc = pd_scores.shape
    n = gt_bboxes.shape[1]

    pdT = jnp.transpose(pd_scores, (0, 2, 1))
    pbT = jnp.transpose(pd_bboxes, (0, 2, 1))
    ancT = jnp.transpose(anc_points, (1, 0))
    lab = gt_labels.reshape(bs, n)

    eps = 1e-7
    atan_pd = jnp.arctan((pd_bboxes[..., 2] - pd_bboxes[..., 0]) /
                         (pd_bboxes[..., 3] - pd_bboxes[..., 1] + eps))
    atan_pd = atan_pd.reshape(bs, 1, na)
    atan_gt = jnp.arctan((gt_bboxes[..., 2] - gt_bboxes[..., 0]) /
                         (gt_bboxes[..., 3] - gt_bboxes[..., 1] + eps))
    atan_gt = atan_gt.reshape(bs, n, 1)

    out_shape = [
        jax.ShapeDtypeStruct((bs, 1, na), jnp.int32),
        jax.ShapeDtypeStruct((bs, 4, na), jnp.float32),
        jax.ShapeDtypeStruct((bs, nc, na), jnp.float32),
        jax.ShapeDtypeStruct((bs, 1, na), jnp.float32),
        jax.ShapeDtypeStruct((bs, 1, na), jnp.int32),
    ]
    grid_spec = pltpu.PrefetchScalarGridSpec(
        num_scalar_prefetch=1,
        grid=(bs,),
        in_specs=[
            pl.BlockSpec((1, nc, na), lambda b, L: (b, 0, 0)),
            pl.BlockSpec((1, 4, na), lambda b, L: (b, 0, 0)),
            pl.BlockSpec((2, na), lambda b, L: (0, 0)),
            pl.BlockSpec((1, n, 4), lambda b, L: (b, 0, 0)),
            pl.BlockSpec((1, n, 1), lambda b, L: (b, 0, 0)),
            pl.BlockSpec((1, n, 1), lambda b, L: (b, 0, 0)),
            pl.BlockSpec((1, 1, na), lambda b, L: (b, 0, 0)),
            pl.BlockSpec((1, n, 1), lambda b, L: (b, 0, 0)),
        ],
        out_specs=[
            pl.BlockSpec((1, 1, na), lambda b, L: (b, 0, 0)),
            pl.BlockSpec((1, 4, na), lambda b, L: (b, 0, 0)),
            pl.BlockSpec((1, nc, na), lambda b, L: (b, 0, 0)),
            pl.BlockSpec((1, 1, na), lambda b, L: (b, 0, 0)),
            pl.BlockSpec((1, 1, na), lambda b, L: (b, 0, 0)),
        ],
        scratch_shapes=[pltpu.VMEM((n, na), jnp.float32)],
    )
    tl3, tb3, ts3, fg3, tgi3 = pl.pallas_call(
        _assigner_body,
        grid_spec=grid_spec,
        out_shape=out_shape,
        compiler_params=pltpu.CompilerParams(
            dimension_semantics=("arbitrary",)),
    )(lab, pdT, pbT, ancT, gt_bboxes, gt_labels, mask_gt, atan_pd, atan_gt)

    target_labels = tl3.reshape(bs, na)
    target_bboxes = jnp.transpose(tb3, (0, 2, 1))
    target_scores = jnp.transpose(ts3, (0, 2, 1))
    fg_mask = fg3.reshape(bs, na).astype(bool)
    target_gt_idx = tgi3.reshape(bs, na)
    return target_labels, target_bboxes, target_scores, fg_mask, target_gt_idx
